# baseline (speedup 1.0000x reference)
import functools

import jax
import jax.numpy as jnp
from jax import lax
from jax.experimental import pallas as pl
from jax.experimental.pallas import tpu as pltpu
from jax.experimental.pallas import tpu_sc as plsc

DIM = 64
SENT_SHAPE = (4096, 200)
_B = SENT_SHAPE[0] * SENT_SHAPE[1]

_info = plsc.get_sparse_core_info()
_NC, _NS = _info.num_cores, _info.num_subcores
_NW = _NC * _NS
_BPW = _B // _NW
_CH = 1024
_NIDX = _CH // 128
_G = _BPW // _CH

_mesh = plsc.VectorSubcoreMesh(core_axis_name="c", subcore_axis_name="s")


@functools.partial(
    pl.kernel,
    mesh=_mesh,
    out_type=jax.ShapeDtypeStruct((_B, DIM), jnp.float32),
    scratch_types=[
        pltpu.VMEM((_NIDX, 128), jnp.int32),
        pltpu.VMEM((_CH, DIM), jnp.float32),
        pltpu.SemaphoreType.DMA,
    ],
    compiler_params=pltpu.CompilerParams(use_tc_tiling_on_sc=False),
)
def _emb_lookup(idx_hbm, table_hbm, out_hbm, idx_v, rows_v, gsem):
    wid = lax.axis_index("s") * _NC + lax.axis_index("c")
    base = wid * _BPW

    def body(g, carry):
        off = pl.multiple_of(base + g * _CH, _CH)
        row0 = pl.multiple_of(wid * (_BPW // 128) + g * _NIDX, _NIDX)
        pltpu.sync_copy(idx_hbm.at[pl.ds(row0, _NIDX)], idx_v)
        copies = [
            pltpu.async_copy(
                table_hbm.at[idx_v.at[j]],
                rows_v.at[pl.ds(j * 128, 128)],
                gsem,
            )
            for j in range(_NIDX)
        ]
        for c in copies:
            c.wait()
        pltpu.sync_copy(rows_v, out_hbm.at[pl.ds(off, _CH)])
        return carry

    lax.fori_loop(0, _G, body, 0)


def kernel(sentence, table):
    idx = sentence.reshape(_B // 128, 128).astype(jnp.int32)
    out = _emb_lookup(idx, table)
    return out.reshape(SENT_SHAPE[0], SENT_SHAPE[1], DIM)


# speedup vs baseline: 1.0048x; 1.0048x over previous
import functools

import jax
import jax.numpy as jnp
from jax import lax
from jax.experimental import pallas as pl
from jax.experimental.pallas import tpu as pltpu
from jax.experimental.pallas import tpu_sc as plsc

DIM = 64
SENT_SHAPE = (4096, 200)
_B = SENT_SHAPE[0] * SENT_SHAPE[1]

_info = plsc.get_sparse_core_info()
_NC, _NS = _info.num_cores, _info.num_subcores
_NW = _NC * _NS
_BPW = _B // _NW
_CH = 512
_NIDX = _CH // 128
_G = _BPW // _CH

_mesh = plsc.VectorSubcoreMesh(core_axis_name="c", subcore_axis_name="s")


@functools.partial(
    pl.kernel,
    mesh=_mesh,
    out_type=jax.ShapeDtypeStruct((_B, DIM), jnp.float32),
    scratch_types=[
        pltpu.VMEM((2, _NIDX, 128), jnp.int32),
        pltpu.VMEM((2, _CH, DIM), jnp.float32),
        pltpu.SemaphoreType.DMA,
        pltpu.SemaphoreType.DMA,
    ],
    compiler_params=pltpu.CompilerParams(use_tc_tiling_on_sc=False),
)
def _emb_lookup(idx_hbm, table_hbm, out_hbm, idx_v, rows_v, sem0, sem1):
    wid = lax.axis_index("s") * _NC + lax.axis_index("c")
    base = wid * _BPW
    chunk0 = wid * _G
    sems = (sem0, sem1)

    def fire(g, b):
        pltpu.sync_copy(idx_hbm.at[chunk0 + g], idx_v.at[b])
        for j in range(_NIDX):
            pltpu.async_copy(
                table_hbm.at[idx_v.at[b, j]],
                rows_v.at[b, pl.ds(j * 128, 128)],
                sems[b],
            )

    def drain(b):
        for j in range(_NIDX):
            pltpu.make_async_copy(
                table_hbm.at[idx_v.at[b, j]],
                rows_v.at[b, pl.ds(j * 128, 128)],
                sems[b],
            ).wait()

    def writeout(g, b):
        off = pl.multiple_of(base + g * _CH, _CH)
        pltpu.sync_copy(rows_v.at[b], out_hbm.at[pl.ds(off, _CH)])

    fire(0, 0)

    def body(p, carry):
        g = 2 * p
        fire(g + 1, 1)
        drain(0)
        writeout(g, 0)

        @pl.when(g + 2 < _G)
        def _():
            fire(g + 2, 0)

        drain(1)
        writeout(g + 1, 1)
        return carry

    lax.fori_loop(0, _G // 2, body, 0)


def kernel(sentence, table):
    idx = sentence.reshape(_B // _CH, _NIDX, 128).astype(jnp.int32)
    out = _emb_lookup(idx, table)
    return out.reshape(SENT_SHAPE[0], SENT_SHAPE[1], DIM)


# speedup vs baseline: 1.0105x; 1.0057x over previous
import functools

import jax
import jax.numpy as jnp
from jax import lax
from jax.experimental import pallas as pl
from jax.experimental.pallas import tpu as pltpu
from jax.experimental.pallas import tpu_sc as plsc

DIM = 64
_S0, _S1 = 4096, 200

_info = plsc.get_sparse_core_info()
_NC, _NS = _info.num_cores, _info.num_subcores
_NW = _NC * _NS
_RPW = _S0 // _NW
_CHR = 4
_G = _RPW // _CHR
_SPLITS = ((0, 96), (96, 104))

_mesh = plsc.VectorSubcoreMesh(core_axis_name="c", subcore_axis_name="s")


@functools.partial(
    pl.kernel,
    mesh=_mesh,
    out_type=jax.ShapeDtypeStruct((_S0, _S1, DIM), jnp.float32),
    scratch_types=[
        pltpu.VMEM((2, _CHR, _S1), jnp.int32),
        pltpu.VMEM((2, _CHR, _S1, DIM), jnp.float32),
        pltpu.SemaphoreType.DMA,
        pltpu.SemaphoreType.DMA,
    ],
    compiler_params=pltpu.CompilerParams(use_tc_tiling_on_sc=False),
)
def _emb_lookup(sent_hbm, table_hbm, out_hbm, idx_v, rows_v, sem0, sem1):
    wid = lax.axis_index("s") * _NC + lax.axis_index("c")
    row0 = wid * _RPW
    sems = (sem0, sem1)

    def fire(g, b):
        sr = pl.multiple_of(row0 + g * _CHR, _CHR)
        pltpu.sync_copy(sent_hbm.at[pl.ds(sr, _CHR)], idx_v.at[b])
        for s in range(_CHR):
            for off, n in _SPLITS:
                pltpu.async_copy(
                    table_hbm.at[idx_v.at[b, s, pl.ds(off, n)]],
                    rows_v.at[b, s, pl.ds(off, n)],
                    sems[b],
                )

    def drain(b):
        for s in range(_CHR):
            for off, n in _SPLITS:
                pltpu.make_async_copy(
                    table_hbm.at[idx_v.at[b, s, pl.ds(off, n)]],
                    rows_v.at[b, s, pl.ds(off, n)],
                    sems[b],
                ).wait()

    def writeout(g, b):
        sr = pl.multiple_of(row0 + g * _CHR, _CHR)
        pltpu.sync_copy(rows_v.at[b], out_hbm.at[pl.ds(sr, _CHR)])

    fire(0, 0)

    def body(p, carry):
        g = 2 * p
        fire(g + 1, 1)
        drain(0)
        writeout(g, 0)

        @pl.when(g + 2 < _G)
        def _():
            fire(g + 2, 0)

        drain(1)
        writeout(g + 1, 1)
        return carry

    lax.fori_loop(0, _G // 2, body, 0)


def kernel(sentence, table):
    return _emb_lookup(sentence.astype(jnp.int32), table)
